# baseline (speedup 1.0000x reference)
import functools

import jax
import jax.numpy as jnp
from jax import lax
from jax.experimental import pallas as pl
from jax.experimental.pallas import tpu as pltpu
from jax.experimental.pallas import tpu_sc as plsc

N = 10000
E = 320000
DIN = 128
D = 64
B = 16

NC = 2
NS = 16
NW = NC * NS
EW = E // NW
CH = 128
NFULL = EW // CH
TAIL = EW - NFULL * CH
RSTRIDE = 624
RWIN = 640
RSEG = 80

_MESH = plsc.VectorSubcoreMesh(core_axis_name="c", subcore_axis_name="s")
_SC_PARAMS = pltpu.CompilerParams(use_tc_tiling_on_sc=False, has_side_effects=True)


_DEGQ = 6

def _deg_body(dstA_hbm, dstB_hbm, degout_hbm, dstv, dstvB, ones_v, zbuf, acc,
              dsem):
    cid = lax.axis_index("c")
    sid = lax.axis_index("s")
    wid = sid * NC + cid

    def fill_z(i, _):
        zbuf[i] = jnp.zeros((16,), jnp.float32)
        return 0

    lax.fori_loop(0, RWIN, fill_z, 0)

    def fill_o(i, _):
        ones_v[i] = jnp.ones((16,), jnp.float32)
        return 0

    lax.fori_loop(0, CH, fill_o, 0)

    pltpu.sync_copy(zbuf, acc.at[pl.ds(sid * RSTRIDE, RWIN)])
    plsc.subcore_barrier()

    pltpu.sync_copy(dstA_hbm.at[wid], dstv)
    pltpu.sync_copy(dstB_hbm.at[wid], dstvB)

    def body(j, _):
        pltpu.sync_copy(ones_v, acc.at[dstv.at[j]], add=True)
        return 0

    lax.fori_loop(0, NFULL, body, 0)
    pltpu.sync_copy(ones_v.at[pl.ds(0, TAIL)], acc.at[dstvB.at[0]], add=True)
    plsc.subcore_barrier()

    pltpu.sync_copy(acc.at[pl.ds(sid * RSTRIDE, RWIN)], zbuf)
    pltpu.sync_copy(zbuf, degout_hbm.at[cid, pl.ds(sid * RSTRIDE, RWIN)])


_deg_call = pl.kernel(
    _deg_body,
    out_type=jax.ShapeDtypeStruct((NC, N, 16), jnp.float32),
    mesh=_MESH,
    compiler_params=_SC_PARAMS,
    scratch_types=[
        pltpu.VMEM((NFULL, CH), jnp.int32),
        pltpu.VMEM((1, TAIL), jnp.int32),
        pltpu.VMEM((CH, 16), jnp.float32),
        pltpu.VMEM((RWIN, 16), jnp.float32),
        pltpu.VMEM_SHARED((N, 16), jnp.float32),
        pltpu.SemaphoreType.DMA,
    ],
)


_NBUF = 2


def _edge_body(hp_hbm, srcA_hbm, srcB_hbm, dstA_hbm, dstB_hbm, pout_hbm,
               srcv, srcvB, dstv, dstvB, rows, trows, zbuf, acc, hps, *sems):
    cid = lax.axis_index("c")
    sid = lax.axis_index("s")
    wid = sid * NC + cid

    for k in range(RWIN // RSEG):
        off = sid * RSTRIDE + k * RSEG
        pltpu.sync_copy(hp_hbm.at[pl.ds(off, RSEG)], zbuf)
        pltpu.sync_copy(zbuf, hps.at[pl.ds(off, RSEG)])

    def fill_z(i, _):
        for k in range(D // 16):
            zbuf[i, pl.ds(k * 16, 16)] = jnp.zeros((16,), jnp.float32)
        return 0

    lax.fori_loop(0, RSEG, fill_z, 0)
    for k in range(RWIN // RSEG):
        pltpu.sync_copy(zbuf, acc.at[pl.ds(sid * RSTRIDE + k * RSEG, RSEG)])
    plsc.subcore_barrier()

    pltpu.sync_copy(srcA_hbm.at[wid], srcv)
    pltpu.sync_copy(srcB_hbm.at[wid], srcvB)
    pltpu.sync_copy(dstA_hbm.at[wid], dstv)
    pltpu.sync_copy(dstB_hbm.at[wid], dstvB)

    def gather(j, b):
        pltpu.async_copy(hps.at[srcv.at[j]], rows.at[b], sems[b])

    def drain(j, b):
        pltpu.make_async_copy(hps.at[srcv.at[j]], rows.at[b],
                              sems[b]).wait()
        pltpu.sync_copy(rows.at[b], acc.at[dstv.at[j]], add=True)

    for b in range(_NBUF - 1):
        gather(b, b)

    def body(i, _):
        j0 = _NBUF * i
        for b in range(_NBUF):
            gather(j0 + b + _NBUF - 1, (b + _NBUF - 1) % _NBUF)
            drain(j0 + b, b)
        return 0

    lax.fori_loop(0, NFULL // _NBUF - 1, body, 0)
    j0 = NFULL - _NBUF
    gather(NFULL - 1, (_NBUF - 1) % _NBUF)
    pltpu.async_copy(hps.at[srcvB.at[0]], trows, sems[_NBUF])
    for b in range(_NBUF):
        drain(j0 + b, b)
    pltpu.make_async_copy(hps.at[srcvB.at[0]], trows, sems[_NBUF]).wait()
    pltpu.sync_copy(trows, acc.at[dstvB.at[0]], add=True)

    plsc.subcore_barrier()
    for k in range(RWIN // RSEG):
        off = sid * RSTRIDE + k * RSEG
        pltpu.sync_copy(acc.at[pl.ds(off, RSEG)], zbuf)
        pltpu.sync_copy(zbuf, pout_hbm.at[cid, pl.ds(off, RSEG)])


_edge_call = pl.kernel(
    _edge_body,
    out_type=jax.ShapeDtypeStruct((NC, N, D), jnp.float32),
    mesh=_MESH,
    compiler_params=_SC_PARAMS,
    scratch_types=[
        pltpu.VMEM((NFULL, CH), jnp.int32),
        pltpu.VMEM((1, TAIL), jnp.int32),
        pltpu.VMEM((NFULL, CH), jnp.int32),
        pltpu.VMEM((1, TAIL), jnp.int32),
        pltpu.VMEM((_NBUF, CH, D), jnp.float32),
        pltpu.VMEM((TAIL, D), jnp.float32),
        pltpu.VMEM((RSEG, D), jnp.float32),
        pltpu.VMEM_SHARED((N, D), jnp.float32),
        pltpu.VMEM_SHARED((N, D), jnp.float32),
    ] + [pltpu.SemaphoreType.DMA] * (_NBUF + 1),
)


_R = 1000
_DOT = dict(preferred_element_type=jnp.float32, precision=lax.Precision.HIGHEST)


def _b_body(x_ref, d0_ref, d1_ref, W_ref, dinv_ref, hp_ref):
    deg = d0_ref[:, :1] + d1_ref[:, :1] + 1.0
    dinv = lax.rsqrt(deg)
    dinv_ref[...] = dinv
    hp_ref[...] = jnp.dot(x_ref[...], W_ref[...], **_DOT) * dinv


def _first_layer(x, deg0, deg1, W0):
    return pl.pallas_call(
        _b_body,
        grid=(N // _R,),
        in_specs=[
            pl.BlockSpec((_R, DIN), lambda i: (i, 0)),
            pl.BlockSpec((_R, 16), lambda i: (i, 0)),
            pl.BlockSpec((_R, 16), lambda i: (i, 0)),
            pl.BlockSpec((DIN, D), lambda i: (0, 0)),
        ],
        out_specs=[
            pl.BlockSpec((_R, 1), lambda i: (i, 0)),
            pl.BlockSpec((_R, D), lambda i: (i, 0)),
        ],
        out_shape=[
            jax.ShapeDtypeStruct((N, 1), jnp.float32),
            jax.ShapeDtypeStruct((N, D), jnp.float32),
        ],
    )(x, deg0, deg1, W0)


def _c_body(p0, p1, hp, dinv, b, W, m_in, hp_next, m_out, *, first):
    out = dinv[...] * (p0[...] + p1[...] + hp[...]) + b[...]
    a = jnp.maximum(out, 0.0)
    m_out[...] = a if first else jnp.maximum(m_in[...], a)
    hp_next[...] = jnp.dot(a, W[...], **_DOT) * dinv[...]


def _mid_layer(p0, p1, hp, dinv, b, W, m_in, first):
    body = functools.partial(_c_body, first=first)
    return pl.pallas_call(
        body,
        grid=(N // _R,),
        in_specs=[
            pl.BlockSpec((_R, D), lambda i: (i, 0)),
            pl.BlockSpec((_R, D), lambda i: (i, 0)),
            pl.BlockSpec((_R, D), lambda i: (i, 0)),
            pl.BlockSpec((_R, 1), lambda i: (i, 0)),
            pl.BlockSpec((1, D), lambda i: (0, 0)),
            pl.BlockSpec((D, D), lambda i: (0, 0)),
            pl.BlockSpec((_R, D), lambda i: (i, 0)),
        ],
        out_specs=[
            pl.BlockSpec((_R, D), lambda i: (i, 0)),
            pl.BlockSpec((_R, D), lambda i: (i, 0)),
        ],
        out_shape=[
            jax.ShapeDtypeStruct((N, D), jnp.float32),
            jax.ShapeDtypeStruct((N, D), jnp.float32),
        ],
    )(p0, p1, hp, dinv, b, W, m_in)


def _tail_body(p0, p1, hp, dinv, b, m_in, batch_ref, y_ref,
               Wm0, bm0, Wm1, bm1, Wm2, bm2, Wm3, bm3,
               pred_ref, loss_ref, pooled):
    out = dinv[...] * (p0[...] + p1[...] + hp[...]) + b[...]
    m = jnp.maximum(m_in[...], out)
    i = pl.program_id(0)
    seg = lax.broadcasted_iota(jnp.int32, (1, B), 1)
    oh = (batch_ref[...] == seg).astype(jnp.float32)
    contrib = lax.dot_general(oh, m, (((0,), (0,)), ((), ())),
                              preferred_element_type=jnp.float32,
                              precision=lax.Precision.HIGHEST)

    @pl.when(i == 0)
    def _():
        pooled[...] = contrib

    @pl.when(i > 0)
    def _():
        pooled[...] = pooled[...] + contrib

    @pl.when(i == pl.num_programs(0) - 1)
    def _():
        h = jnp.maximum(jnp.dot(pooled[...], Wm0[...], **_DOT) + bm0[...], 0.0)
        h = jnp.maximum(jnp.dot(h, Wm1[...], **_DOT) + bm1[...], 0.0)
        h = jnp.maximum(jnp.dot(h, Wm2[...], **_DOT) + bm2[...], 0.0)
        pred = jnp.dot(h, Wm3[...], **_DOT) + bm3[...]
        pred_ref[...] = pred
        loss_ref[...] = (jnp.sum((pred - y_ref[...]) ** 2) / B).reshape(1, 1)


def _tail_layer(p0, p1, hp, dinv, b, m_in, batch2d, y,
                Wm0, bm0, Wm1, bm1, Wm2, bm2, Wm3, bm3):
    full = lambda a, bb: pl.BlockSpec((a, bb), lambda i: (0, 0))
    return pl.pallas_call(
        _tail_body,
        grid=(N // _R,),
        in_specs=[
            pl.BlockSpec((_R, D), lambda i: (i, 0)),
            pl.BlockSpec((_R, D), lambda i: (i, 0)),
            pl.BlockSpec((_R, D), lambda i: (i, 0)),
            pl.BlockSpec((_R, 1), lambda i: (i, 0)),
            full(1, D),
            pl.BlockSpec((_R, D), lambda i: (i, 0)),
            pl.BlockSpec((_R, 1), lambda i: (i, 0)),
            full(B, 1),
            full(D, 32), full(1, 32),
            full(32, 16), full(1, 16),
            full(16, 8), full(1, 8),
            full(8, 1), full(1, 1),
        ],
        out_specs=[full(B, 1), full(1, 1)],
        out_shape=[
            jax.ShapeDtypeStruct((B, 1), jnp.float32),
            jax.ShapeDtypeStruct((1, 1), jnp.float32),
        ],
        scratch_shapes=[pltpu.VMEM((B, D), jnp.float32)],
    )(p0, p1, hp, dinv, b, m_in, batch2d, y,
      Wm0, bm0, Wm1, bm1, Wm2, bm2, Wm3, bm3)


def kernel(x, edge_index, edge_attr, batch, y,
           W0, b0, W1, b1, W2, b2, W3, b3, W4, b4, W5, b5,
           Wm0, bm0, Wm1, bm1, Wm2, bm2, Wm3, bm3):
    srcw = edge_index[0].reshape(NW, EW)
    dstw = edge_index[1].reshape(NW, EW)
    srcA = srcw[:, :NFULL * CH].reshape(NW, NFULL, CH)
    srcB = srcw[:, NFULL * CH:].reshape(NW, 1, TAIL)
    dstA = dstw[:, :NFULL * CH].reshape(NW, NFULL, CH)
    dstB = dstw[:, NFULL * CH:].reshape(NW, 1, TAIL)

    deg_p = _deg_call(dstA, dstB)
    dinv, hp = _first_layer(x, deg_p[0], deg_p[1], W0)

    Ws = [W1, W2, W3, W4, W5]
    bs = [b0, b1, b2, b3, b4, b5]
    m = None
    for l in range(6):
        P = _edge_call(hp, srcA, srcB, dstA, dstB)
        brow = bs[l].reshape(1, D)
        if l == 0:
            hp, m = _mid_layer(P[0], P[1], hp, dinv, brow, Ws[0], hp,
                               first=True)
        elif l < 5:
            hp, m = _mid_layer(P[0], P[1], hp, dinv, brow, Ws[l], m,
                               first=False)
        else:
            pred, loss = _tail_layer(
                P[0], P[1], hp, dinv, brow, m, batch.reshape(N, 1), y,
                Wm0, bm0.reshape(1, 32), Wm1, bm1.reshape(1, 16),
                Wm2, bm2.reshape(1, 8), Wm3, bm3.reshape(1, 1))
    return pred, loss.reshape(())


# speedup vs baseline: 1.3113x; 1.3113x over previous
import functools

import jax
import jax.numpy as jnp
from jax import lax
from jax.experimental import pallas as pl
from jax.experimental.pallas import tpu as pltpu
from jax.experimental.pallas import tpu_sc as plsc

N = 10000
E = 320000
DIN = 128
D = 64
B = 16

NC = 2
NS = 16
NW = NC * NS
EW = E // NW
CH = 128
NFULL = EW // CH
TAIL = EW - NFULL * CH
RSTRIDE = 624
RWIN = 640
RSEG = 160

_MESH = plsc.VectorSubcoreMesh(core_axis_name="c", subcore_axis_name="s")
_SC_PARAMS = pltpu.CompilerParams(use_tc_tiling_on_sc=False, has_side_effects=True)


_DEGQ = 6

def _deg_body(dstA_hbm, dstB_hbm, degout_hbm, dstv, dstvB, ones_v, zbuf, acc,
              dsem):
    cid = lax.axis_index("c")
    sid = lax.axis_index("s")
    wid = sid * NC + cid

    def fill_z(i, _):
        zbuf[i] = jnp.zeros((16,), jnp.float32)
        return 0

    lax.fori_loop(0, RWIN, fill_z, 0)

    def fill_o(i, _):
        ones_v[i] = jnp.ones((16,), jnp.float32)
        return 0

    lax.fori_loop(0, CH, fill_o, 0)

    pltpu.sync_copy(zbuf, acc.at[pl.ds(sid * RSTRIDE, RWIN)])
    plsc.subcore_barrier()

    pltpu.sync_copy(dstA_hbm.at[wid], dstv)
    pltpu.sync_copy(dstB_hbm.at[wid], dstvB)

    def body(j, _):
        pltpu.sync_copy(ones_v, acc.at[dstv.at[j]], add=True)
        return 0

    lax.fori_loop(0, NFULL, body, 0)
    pltpu.sync_copy(ones_v.at[pl.ds(0, TAIL)], acc.at[dstvB.at[0]], add=True)
    plsc.subcore_barrier()

    pltpu.sync_copy(acc.at[pl.ds(sid * RSTRIDE, RWIN)], zbuf)
    pltpu.sync_copy(zbuf, degout_hbm.at[cid, pl.ds(sid * RSTRIDE, RWIN)])


_deg_call = pl.kernel(
    _deg_body,
    out_type=jax.ShapeDtypeStruct((NC, N, 16), jnp.float32),
    mesh=_MESH,
    compiler_params=_SC_PARAMS,
    scratch_types=[
        pltpu.VMEM((NFULL, CH), jnp.int32),
        pltpu.VMEM((1, TAIL), jnp.int32),
        pltpu.VMEM((CH, 16), jnp.float32),
        pltpu.VMEM((RWIN, 16), jnp.float32),
        pltpu.VMEM_SHARED((N, 16), jnp.float32),
        pltpu.SemaphoreType.DMA,
    ],
)


_NBUF = 3


def _edge_body(hp_hbm, srcA_hbm, srcB_hbm, dstA_hbm, dstB_hbm, pout_hbm,
               srcv, srcvB, dstv, dstvB, rows, trows, zbuf, acc, *sems):
    cid = lax.axis_index("c")
    sid = lax.axis_index("s")
    wid = sid * NC + cid

    def fill_z(i, _):
        for k in range(D // 16):
            zbuf[i, pl.ds(k * 16, 16)] = jnp.zeros((16,), jnp.float32)
        return 0

    lax.fori_loop(0, RSEG, fill_z, 0)
    for k in range(RWIN // RSEG):
        pltpu.sync_copy(zbuf, acc.at[pl.ds(sid * RSTRIDE + k * RSEG, RSEG)])
    plsc.subcore_barrier()

    pltpu.sync_copy(srcA_hbm.at[wid], srcv)
    pltpu.sync_copy(srcB_hbm.at[wid], srcvB)
    pltpu.sync_copy(dstA_hbm.at[wid], dstv)
    pltpu.sync_copy(dstB_hbm.at[wid], dstvB)

    def gather(j, b):
        pltpu.async_copy(hp_hbm.at[srcv.at[j]], rows.at[b], sems[b])

    def drain(j, b):
        pltpu.make_async_copy(hp_hbm.at[srcv.at[j]], rows.at[b],
                              sems[b]).wait()
        pltpu.sync_copy(rows.at[b], acc.at[dstv.at[j]], add=True)

    for b in range(_NBUF - 1):
        gather(b, b)

    def body(i, _):
        j0 = _NBUF * i
        for b in range(_NBUF):
            gather(j0 + b + _NBUF - 1, (b + _NBUF - 1) % _NBUF)
            drain(j0 + b, b)
        return 0

    lax.fori_loop(0, NFULL // _NBUF - 1, body, 0)
    j0 = NFULL - _NBUF
    gather(NFULL - 1, (_NBUF - 1) % _NBUF)
    pltpu.async_copy(hp_hbm.at[srcvB.at[0]], trows, sems[_NBUF])
    for b in range(_NBUF):
        drain(j0 + b, b)
    pltpu.make_async_copy(hp_hbm.at[srcvB.at[0]], trows, sems[_NBUF]).wait()
    pltpu.sync_copy(trows, acc.at[dstvB.at[0]], add=True)

    plsc.subcore_barrier()
    for k in range(RWIN // RSEG):
        off = sid * RSTRIDE + k * RSEG
        pltpu.sync_copy(acc.at[pl.ds(off, RSEG)], zbuf)
        pltpu.sync_copy(zbuf, pout_hbm.at[cid, pl.ds(off, RSEG)])


_edge_call = pl.kernel(
    _edge_body,
    out_type=jax.ShapeDtypeStruct((NC, N, D), jnp.float32),
    mesh=_MESH,
    compiler_params=_SC_PARAMS,
    scratch_types=[
        pltpu.VMEM((NFULL, CH), jnp.int32),
        pltpu.VMEM((1, TAIL), jnp.int32),
        pltpu.VMEM((NFULL, CH), jnp.int32),
        pltpu.VMEM((1, TAIL), jnp.int32),
        pltpu.VMEM((_NBUF, CH, D), jnp.float32),
        pltpu.VMEM((TAIL, D), jnp.float32),
        pltpu.VMEM((RSEG, D), jnp.float32),
        pltpu.VMEM_SHARED((N, D), jnp.float32),
    ] + [pltpu.SemaphoreType.DMA] * (_NBUF + 1),
)


_R = 1000
_DOT = dict(preferred_element_type=jnp.float32, precision=lax.Precision.HIGHEST)


def _b_body(x_ref, d0_ref, d1_ref, W_ref, dinv_ref, hp_ref):
    deg = d0_ref[:, :1] + d1_ref[:, :1] + 1.0
    dinv = lax.rsqrt(deg)
    dinv_ref[...] = dinv
    hp_ref[...] = jnp.dot(x_ref[...], W_ref[...], **_DOT) * dinv


def _first_layer(x, deg0, deg1, W0):
    return pl.pallas_call(
        _b_body,
        grid=(N // _R,),
        in_specs=[
            pl.BlockSpec((_R, DIN), lambda i: (i, 0)),
            pl.BlockSpec((_R, 16), lambda i: (i, 0)),
            pl.BlockSpec((_R, 16), lambda i: (i, 0)),
            pl.BlockSpec((DIN, D), lambda i: (0, 0)),
        ],
        out_specs=[
            pl.BlockSpec((_R, 1), lambda i: (i, 0)),
            pl.BlockSpec((_R, D), lambda i: (i, 0)),
        ],
        out_shape=[
            jax.ShapeDtypeStruct((N, 1), jnp.float32),
            jax.ShapeDtypeStruct((N, D), jnp.float32),
        ],
    )(x, deg0, deg1, W0)


def _c_body(p0, p1, hp, dinv, b, W, m_in, hp_next, m_out, *, first):
    out = dinv[...] * (p0[...] + p1[...] + hp[...]) + b[...]
    a = jnp.maximum(out, 0.0)
    m_out[...] = a if first else jnp.maximum(m_in[...], a)
    hp_next[...] = jnp.dot(a, W[...], **_DOT) * dinv[...]


def _mid_layer(p0, p1, hp, dinv, b, W, m_in, first):
    body = functools.partial(_c_body, first=first)
    return pl.pallas_call(
        body,
        grid=(N // _R,),
        in_specs=[
            pl.BlockSpec((_R, D), lambda i: (i, 0)),
            pl.BlockSpec((_R, D), lambda i: (i, 0)),
            pl.BlockSpec((_R, D), lambda i: (i, 0)),
            pl.BlockSpec((_R, 1), lambda i: (i, 0)),
            pl.BlockSpec((1, D), lambda i: (0, 0)),
            pl.BlockSpec((D, D), lambda i: (0, 0)),
            pl.BlockSpec((_R, D), lambda i: (i, 0)),
        ],
        out_specs=[
            pl.BlockSpec((_R, D), lambda i: (i, 0)),
            pl.BlockSpec((_R, D), lambda i: (i, 0)),
        ],
        out_shape=[
            jax.ShapeDtypeStruct((N, D), jnp.float32),
            jax.ShapeDtypeStruct((N, D), jnp.float32),
        ],
    )(p0, p1, hp, dinv, b, W, m_in)


def _tail_body(p0, p1, hp, dinv, b, m_in, batch_ref, y_ref,
               Wm0, bm0, Wm1, bm1, Wm2, bm2, Wm3, bm3,
               pred_ref, loss_ref, pooled):
    out = dinv[...] * (p0[...] + p1[...] + hp[...]) + b[...]
    m = jnp.maximum(m_in[...], out)
    i = pl.program_id(0)
    seg = lax.broadcasted_iota(jnp.int32, (1, B), 1)
    oh = (batch_ref[...] == seg).astype(jnp.float32)
    contrib = lax.dot_general(oh, m, (((0,), (0,)), ((), ())),
                              preferred_element_type=jnp.float32,
                              precision=lax.Precision.HIGHEST)

    @pl.when(i == 0)
    def _():
        pooled[...] = contrib

    @pl.when(i > 0)
    def _():
        pooled[...] = pooled[...] + contrib

    @pl.when(i == pl.num_programs(0) - 1)
    def _():
        h = jnp.maximum(jnp.dot(pooled[...], Wm0[...], **_DOT) + bm0[...], 0.0)
        h = jnp.maximum(jnp.dot(h, Wm1[...], **_DOT) + bm1[...], 0.0)
        h = jnp.maximum(jnp.dot(h, Wm2[...], **_DOT) + bm2[...], 0.0)
        pred = jnp.dot(h, Wm3[...], **_DOT) + bm3[...]
        pred_ref[...] = pred
        loss_ref[...] = (jnp.sum((pred - y_ref[...]) ** 2) / B).reshape(1, 1)


def _tail_layer(p0, p1, hp, dinv, b, m_in, batch2d, y,
                Wm0, bm0, Wm1, bm1, Wm2, bm2, Wm3, bm3):
    full = lambda a, bb: pl.BlockSpec((a, bb), lambda i: (0, 0))
    return pl.pallas_call(
        _tail_body,
        grid=(N // _R,),
        in_specs=[
            pl.BlockSpec((_R, D), lambda i: (i, 0)),
            pl.BlockSpec((_R, D), lambda i: (i, 0)),
            pl.BlockSpec((_R, D), lambda i: (i, 0)),
            pl.BlockSpec((_R, 1), lambda i: (i, 0)),
            full(1, D),
            pl.BlockSpec((_R, D), lambda i: (i, 0)),
            pl.BlockSpec((_R, 1), lambda i: (i, 0)),
            full(B, 1),
            full(D, 32), full(1, 32),
            full(32, 16), full(1, 16),
            full(16, 8), full(1, 8),
            full(8, 1), full(1, 1),
        ],
        out_specs=[full(B, 1), full(1, 1)],
        out_shape=[
            jax.ShapeDtypeStruct((B, 1), jnp.float32),
            jax.ShapeDtypeStruct((1, 1), jnp.float32),
        ],
        scratch_shapes=[pltpu.VMEM((B, D), jnp.float32)],
    )(p0, p1, hp, dinv, b, m_in, batch2d, y,
      Wm0, bm0, Wm1, bm1, Wm2, bm2, Wm3, bm3)


def kernel(x, edge_index, edge_attr, batch, y,
           W0, b0, W1, b1, W2, b2, W3, b3, W4, b4, W5, b5,
           Wm0, bm0, Wm1, bm1, Wm2, bm2, Wm3, bm3):
    srcw = edge_index[0].reshape(NW, EW)
    dstw = edge_index[1].reshape(NW, EW)
    srcA = srcw[:, :NFULL * CH].reshape(NW, NFULL, CH)
    srcB = srcw[:, NFULL * CH:].reshape(NW, 1, TAIL)
    dstA = dstw[:, :NFULL * CH].reshape(NW, NFULL, CH)
    dstB = dstw[:, NFULL * CH:].reshape(NW, 1, TAIL)

    deg_p = _deg_call(dstA, dstB)
    dinv, hp = _first_layer(x, deg_p[0], deg_p[1], W0)

    Ws = [W1, W2, W3, W4, W5]
    bs = [b0, b1, b2, b3, b4, b5]
    m = None
    for l in range(6):
        P = _edge_call(hp, srcA, srcB, dstA, dstB)
        brow = bs[l].reshape(1, D)
        if l == 0:
            hp, m = _mid_layer(P[0], P[1], hp, dinv, brow, Ws[0], hp,
                               first=True)
        elif l < 5:
            hp, m = _mid_layer(P[0], P[1], hp, dinv, brow, Ws[l], m,
                               first=False)
        else:
            pred, loss = _tail_layer(
                P[0], P[1], hp, dinv, brow, m, batch.reshape(N, 1), y,
                Wm0, bm0.reshape(1, 32), Wm1, bm1.reshape(1, 16),
                Wm2, bm2.reshape(1, 8), Wm3, bm3.reshape(1, 1))
    return pred, loss.reshape(())


# speedup vs baseline: 1.4244x; 1.0863x over previous
import functools

import jax
import jax.numpy as jnp
from jax import lax
from jax.experimental import pallas as pl
from jax.experimental.pallas import tpu as pltpu
from jax.experimental.pallas import tpu_sc as plsc

N = 10000
E = 320000
DIN = 128
D = 64
B = 16

NC = 2
NS = 16
NW = NC * NS
EW = E // NW
CH = 128
NFULL = EW // CH
TAIL = EW - NFULL * CH
RSTRIDE = 624
RWIN = 640
RSEG = 160

_MESH = plsc.VectorSubcoreMesh(core_axis_name="c", subcore_axis_name="s")
_SC_PARAMS = pltpu.CompilerParams(use_tc_tiling_on_sc=False, has_side_effects=True)


_DEGQ = 6

def _deg_body(dstA_hbm, dstB_hbm, degout_hbm, dstv, dstvB, ones_v, zbuf, acc,
              dsem):
    cid = lax.axis_index("c")
    sid = lax.axis_index("s")
    wid = sid * NC + cid

    def fill_z(i, _):
        zbuf[i] = jnp.zeros((16,), jnp.float32)
        return 0

    lax.fori_loop(0, RWIN, fill_z, 0)

    def fill_o(i, _):
        ones_v[i] = jnp.ones((16,), jnp.float32)
        return 0

    lax.fori_loop(0, CH, fill_o, 0)

    pltpu.sync_copy(zbuf, acc.at[pl.ds(sid * RSTRIDE, RWIN)])
    plsc.subcore_barrier()

    pltpu.sync_copy(dstA_hbm.at[wid], dstv)
    pltpu.sync_copy(dstB_hbm.at[wid], dstvB)

    def body(j, _):
        pltpu.sync_copy(ones_v, acc.at[dstv.at[j]], add=True)
        return 0

    lax.fori_loop(0, NFULL, body, 0)
    pltpu.sync_copy(ones_v.at[pl.ds(0, TAIL)], acc.at[dstvB.at[0]], add=True)
    plsc.subcore_barrier()

    pltpu.sync_copy(acc.at[pl.ds(sid * RSTRIDE, RWIN)], zbuf)
    pltpu.sync_copy(zbuf, degout_hbm.at[cid, pl.ds(sid * RSTRIDE, RWIN)])


_deg_call = pl.kernel(
    _deg_body,
    out_type=jax.ShapeDtypeStruct((NC, N, 16), jnp.float32),
    mesh=_MESH,
    compiler_params=_SC_PARAMS,
    scratch_types=[
        pltpu.VMEM((NFULL, CH), jnp.int32),
        pltpu.VMEM((1, TAIL), jnp.int32),
        pltpu.VMEM((CH, 16), jnp.float32),
        pltpu.VMEM((RWIN, 16), jnp.float32),
        pltpu.VMEM_SHARED((N, 16), jnp.float32),
        pltpu.SemaphoreType.DMA,
    ],
)


_NBUF = 3


def _edge_body(hp_hbm, srcA_hbm, srcB_hbm, dstA_hbm, dstB_hbm, pout_hbm,
               srcv, srcvB, dstv, dstvB, rows, trows, zbuf, acc, *sems):
    cid = lax.axis_index("c")
    sid = lax.axis_index("s")
    wid = sid * NC + cid

    def fill_z(i, _):
        for k in range(D // 16):
            zbuf[i, pl.ds(k * 16, 16)] = jnp.zeros((16,), jnp.float32)
        return 0

    lax.fori_loop(0, RSEG, fill_z, 0)
    for k in range(RWIN // RSEG):
        pltpu.sync_copy(zbuf, acc.at[pl.ds(sid * RSTRIDE + k * RSEG, RSEG)])
    plsc.subcore_barrier()

    pltpu.sync_copy(srcA_hbm.at[wid], srcv)
    pltpu.sync_copy(srcB_hbm.at[wid], srcvB)
    pltpu.sync_copy(dstA_hbm.at[wid], dstv)
    pltpu.sync_copy(dstB_hbm.at[wid], dstvB)

    def gather(j, b):
        pltpu.async_copy(hp_hbm.at[srcv.at[j]], rows.at[b], sems[b])

    def drain(j, b):
        pltpu.make_async_copy(hp_hbm.at[srcv.at[j]], rows.at[b],
                              sems[b]).wait()
        pltpu.sync_copy(rows.at[b], acc.at[dstv.at[j]], add=True)

    for b in range(_NBUF - 1):
        gather(b, b)

    def body(i, _):
        j0 = _NBUF * i
        for b in range(_NBUF):
            gather(j0 + b + _NBUF - 1, (b + _NBUF - 1) % _NBUF)
            drain(j0 + b, b)
        return 0

    lax.fori_loop(0, NFULL // _NBUF - 1, body, 0)
    j0 = NFULL - _NBUF
    gather(NFULL - 1, (_NBUF - 1) % _NBUF)
    pltpu.async_copy(hp_hbm.at[srcvB.at[0]], trows, sems[_NBUF])
    for b in range(_NBUF):
        drain(j0 + b, b)
    pltpu.make_async_copy(hp_hbm.at[srcvB.at[0]], trows, sems[_NBUF]).wait()
    pltpu.sync_copy(trows, acc.at[dstvB.at[0]], add=True)

    plsc.subcore_barrier()
    for k in range(RWIN // RSEG):
        off = sid * RSTRIDE + k * RSEG
        pltpu.sync_copy(acc.at[pl.ds(off, RSEG)], zbuf)
        pltpu.sync_copy(zbuf, pout_hbm.at[cid, pl.ds(off, RSEG)])


_edge_call = pl.kernel(
    _edge_body,
    out_type=jax.ShapeDtypeStruct((NC, N, D), jnp.float32),
    mesh=_MESH,
    compiler_params=_SC_PARAMS,
    scratch_types=[
        pltpu.VMEM((NFULL, CH), jnp.int32),
        pltpu.VMEM((1, TAIL), jnp.int32),
        pltpu.VMEM((NFULL, CH), jnp.int32),
        pltpu.VMEM((1, TAIL), jnp.int32),
        pltpu.VMEM((_NBUF, CH, D), jnp.float32),
        pltpu.VMEM((TAIL, D), jnp.float32),
        pltpu.VMEM((RSEG, D), jnp.float32),
        pltpu.VMEM_SHARED((N, D), jnp.float32),
    ] + [pltpu.SemaphoreType.DMA] * (_NBUF + 1),
)


_R = 1000
_DOT = dict(preferred_element_type=jnp.float32, precision=lax.Precision.HIGHEST)


def _b_body(x_ref, dp_ref, W_ref, dinv_ref, hp_ref):
    deg = dp_ref[0][:, :1] + dp_ref[1][:, :1] + 1.0
    dinv = lax.rsqrt(deg)
    dinv_ref[...] = dinv
    hp_ref[...] = jnp.dot(x_ref[...], W_ref[...], **_DOT) * dinv


def _first_layer(x, deg_p, W0):
    return pl.pallas_call(
        _b_body,
        grid=(N // _R,),
        in_specs=[
            pl.BlockSpec((_R, DIN), lambda i: (i, 0)),
            pl.BlockSpec((NC, _R, 16), lambda i: (0, i, 0)),
            pl.BlockSpec((DIN, D), lambda i: (0, 0)),
        ],
        out_specs=[
            pl.BlockSpec((_R, 1), lambda i: (i, 0)),
            pl.BlockSpec((_R, D), lambda i: (i, 0)),
        ],
        out_shape=[
            jax.ShapeDtypeStruct((N, 1), jnp.float32),
            jax.ShapeDtypeStruct((N, D), jnp.float32),
        ],
    )(x, deg_p, W0)


def _c_body(p, hp, dinv, b, W, m_in, hp_next, m_out, *, first):
    out = dinv[...] * (p[0] + p[1] + hp[...]) + b[...]
    a = jnp.maximum(out, 0.0)
    m_out[...] = a if first else jnp.maximum(m_in[...], a)
    hp_next[...] = jnp.dot(a, W[...], **_DOT) * dinv[...]


def _mid_layer(p, hp, dinv, b, W, m_in, first):
    body = functools.partial(_c_body, first=first)
    return pl.pallas_call(
        body,
        grid=(N // _R,),
        in_specs=[
            pl.BlockSpec((NC, _R, D), lambda i: (0, i, 0)),
            pl.BlockSpec((_R, D), lambda i: (i, 0)),
            pl.BlockSpec((_R, 1), lambda i: (i, 0)),
            pl.BlockSpec((1, D), lambda i: (0, 0)),
            pl.BlockSpec((D, D), lambda i: (0, 0)),
            pl.BlockSpec((_R, D), lambda i: (i, 0)),
        ],
        out_specs=[
            pl.BlockSpec((_R, D), lambda i: (i, 0)),
            pl.BlockSpec((_R, D), lambda i: (i, 0)),
        ],
        out_shape=[
            jax.ShapeDtypeStruct((N, D), jnp.float32),
            jax.ShapeDtypeStruct((N, D), jnp.float32),
        ],
    )(p, hp, dinv, b, W, m_in)


def _tail_body(p, hp, dinv, b, m_in, batch_ref, y_ref,
               Wm0, bm0, Wm1, bm1, Wm2, bm2, Wm3, bm3,
               pred_ref, loss_ref, pooled):
    out = dinv[...] * (p[0] + p[1] + hp[...]) + b[...]
    m = jnp.maximum(m_in[...], out)
    i = pl.program_id(0)
    seg = lax.broadcasted_iota(jnp.int32, (1, B), 1)
    oh = (batch_ref[...] == seg).astype(jnp.float32)
    contrib = lax.dot_general(oh, m, (((0,), (0,)), ((), ())),
                              preferred_element_type=jnp.float32,
                              precision=lax.Precision.HIGHEST)

    @pl.when(i == 0)
    def _():
        pooled[...] = contrib

    @pl.when(i > 0)
    def _():
        pooled[...] = pooled[...] + contrib

    @pl.when(i == pl.num_programs(0) - 1)
    def _():
        h = jnp.maximum(jnp.dot(pooled[...], Wm0[...], **_DOT) + bm0[...], 0.0)
        h = jnp.maximum(jnp.dot(h, Wm1[...], **_DOT) + bm1[...], 0.0)
        h = jnp.maximum(jnp.dot(h, Wm2[...], **_DOT) + bm2[...], 0.0)
        pred = jnp.dot(h, Wm3[...], **_DOT) + bm3[...]
        pred_ref[...] = pred
        loss_ref[...] = (jnp.sum((pred - y_ref[...]) ** 2) / B).reshape(1, 1)


def _tail_layer(p, hp, dinv, b, m_in, batch2d, y,
                Wm0, bm0, Wm1, bm1, Wm2, bm2, Wm3, bm3):
    full = lambda a, bb: pl.BlockSpec((a, bb), lambda i: (0, 0))
    return pl.pallas_call(
        _tail_body,
        grid=(N // _R,),
        in_specs=[
            pl.BlockSpec((NC, _R, D), lambda i: (0, i, 0)),
            pl.BlockSpec((_R, D), lambda i: (i, 0)),
            pl.BlockSpec((_R, 1), lambda i: (i, 0)),
            full(1, D),
            pl.BlockSpec((_R, D), lambda i: (i, 0)),
            pl.BlockSpec((_R, 1), lambda i: (i, 0)),
            full(B, 1),
            full(D, 32), full(1, 32),
            full(32, 16), full(1, 16),
            full(16, 8), full(1, 8),
            full(8, 1), full(1, 1),
        ],
        out_specs=[full(B, 1), full(1, 1)],
        out_shape=[
            jax.ShapeDtypeStruct((B, 1), jnp.float32),
            jax.ShapeDtypeStruct((1, 1), jnp.float32),
        ],
        scratch_shapes=[pltpu.VMEM((B, D), jnp.float32)],
    )(p, hp, dinv, b, m_in, batch2d, y,
      Wm0, bm0, Wm1, bm1, Wm2, bm2, Wm3, bm3)


def kernel(x, edge_index, edge_attr, batch, y,
           W0, b0, W1, b1, W2, b2, W3, b3, W4, b4, W5, b5,
           Wm0, bm0, Wm1, bm1, Wm2, bm2, Wm3, bm3):
    srcw = edge_index[0].reshape(NW, EW)
    dstw = edge_index[1].reshape(NW, EW)
    srcA = srcw[:, :NFULL * CH].reshape(NW, NFULL, CH)
    srcB = srcw[:, NFULL * CH:].reshape(NW, 1, TAIL)
    dstA = dstw[:, :NFULL * CH].reshape(NW, NFULL, CH)
    dstB = dstw[:, NFULL * CH:].reshape(NW, 1, TAIL)

    deg_p = _deg_call(dstA, dstB)
    dinv, hp = _first_layer(x, deg_p, W0)

    Ws = [W1, W2, W3, W4, W5]
    bs = [b0, b1, b2, b3, b4, b5]
    m = None
    for l in range(6):
        P = _edge_call(hp, srcA, srcB, dstA, dstB)
        brow = bs[l].reshape(1, D)
        if l == 0:
            hp, m = _mid_layer(P, hp, dinv, brow, Ws[0], hp, first=True)
        elif l < 5:
            hp, m = _mid_layer(P, hp, dinv, brow, Ws[l], m, first=False)
        else:
            pred, loss = _tail_layer(
                P, hp, dinv, brow, m, batch.reshape(N, 1), y,
                Wm0, bm0.reshape(1, 32), Wm1, bm1.reshape(1, 16),
                Wm2, bm2.reshape(1, 8), Wm3, bm3.reshape(1, 1))
    return pred, loss.reshape(())
